# baseline (speedup 1.0000x reference)
import functools

import jax
import jax.numpy as jnp
from jax.experimental import pallas as pl
from jax.experimental.pallas import tpu as pltpu

N_LAYERS = 3
SCORE_TILES = 8


def _gcn_kernel(a_ref, c_ref, d_ref, w_ref, circ_out, dis_out, score_out,
                pc_s, pd_s):
    i = pl.program_id(0)

    @pl.when(i == 0)
    def _propagate():
        a = a_ref[:]
        ego_c = c_ref[:]
        ego_d = d_ref[:]

        deg_c = jnp.sum(a, axis=1, keepdims=True)
        deg_d = jnp.sum(a, axis=0, keepdims=True).T
        dc = jnp.where(deg_c > 0, jax.lax.rsqrt(deg_c), 0.0)
        dd = jnp.where(deg_d > 0, jax.lax.rsqrt(deg_d), 0.0)

        def cos_weight(y, ego):
            num = jnp.sum(y * ego, axis=1, keepdims=True)
            ny = jnp.sqrt(jnp.sum(y * y, axis=1, keepdims=True))
            ne = jnp.sqrt(jnp.sum(ego * ego, axis=1, keepdims=True))
            return num / jnp.maximum(ny * ne, 1e-8)

        xc, xd = ego_c, ego_d
        acc_c = jnp.zeros_like(ego_c)
        acc_d = jnp.zeros_like(ego_d)
        for _ in range(N_LAYERS):
            yc = dc * jax.lax.dot(a, dd * xd,
                                  preferred_element_type=jnp.float32)
            yd = dd * jax.lax.dot_general(
                a, dc * xc, (((0,), (0,)), ((), ())),
                preferred_element_type=jnp.float32)
            xc = cos_weight(yc, ego_c) * yc
            xd = cos_weight(yd, ego_d) * yd
            acc_c = acc_c + xc
            acc_d = acc_d + xd

        circ_out[:] = acc_c
        dis_out[:] = acc_d
        pc_s[:] = jax.lax.dot(acc_c, w_ref[:],
                              preferred_element_type=jnp.float32)
        pd_s[:] = acc_d

    rows = pc_s.shape[0] // SCORE_TILES
    tile = pc_s[pl.ds(i * rows, rows), :]
    score_out[:] = jax.lax.dot_general(
        tile, pd_s[:], (((1,), (1,)), ((), ())),
        preferred_element_type=jnp.float32)


@functools.partial(jax.jit)
def kernel(A, circ_emb, dis_emb, re_CD):
    C, D = A.shape
    L = circ_emb.shape[1]
    out_shapes = (
        jax.ShapeDtypeStruct((C, L), jnp.float32),
        jax.ShapeDtypeStruct((D, L), jnp.float32),
        jax.ShapeDtypeStruct((C, D), jnp.float32),
    )
    tile_c = C // SCORE_TILES
    return pl.pallas_call(
        _gcn_kernel,
        grid=(SCORE_TILES,),
        in_specs=[
            pl.BlockSpec((C, D), lambda i: (0, 0)),
            pl.BlockSpec((C, L), lambda i: (0, 0)),
            pl.BlockSpec((D, L), lambda i: (0, 0)),
            pl.BlockSpec((L, L), lambda i: (0, 0)),
        ],
        out_specs=(
            pl.BlockSpec((C, L), lambda i: (0, 0)),
            pl.BlockSpec((D, L), lambda i: (0, 0)),
            pl.BlockSpec((tile_c, D), lambda i: (i, 0)),
        ),
        scratch_shapes=[
            pltpu.VMEM((C, L), jnp.float32),
            pltpu.VMEM((D, L), jnp.float32),
        ],
        out_shape=out_shapes,
        compiler_params=pltpu.CompilerParams(
            vmem_limit_bytes=100 * 1024 * 1024,
        ),
    )(A, circ_emb, dis_emb, re_CD)


# speedup vs baseline: 1.0553x; 1.0553x over previous
import functools

import jax
import jax.numpy as jnp
from jax.experimental import pallas as pl
from jax.experimental.pallas import tpu as pltpu

N_LAYERS = 3


def _gcn_kernel(a_ref, c_ref, d_ref, w_ref, circ_out, dis_out, score_out):
    a = a_ref[:]
    C, D = a.shape
    ego_cT = jnp.transpose(c_ref[:])
    ego_dT = jnp.transpose(d_ref[:])

    deg_c = jax.lax.dot_general(
        jnp.ones((1, D), jnp.float32), a, (((1,), (1,)), ((), ())),
        preferred_element_type=jnp.float32)
    deg_d = jax.lax.dot_general(
        jnp.ones((1, C), jnp.float32), a, (((1,), (0,)), ((), ())),
        preferred_element_type=jnp.float32)
    dc = jnp.where(deg_c > 0, jax.lax.rsqrt(deg_c), 0.0)
    dd = jnp.where(deg_d > 0, jax.lax.rsqrt(deg_d), 0.0)

    def cos_weight(yT, egoT):
        num = jnp.sum(yT * egoT, axis=0, keepdims=True)
        ny = jnp.sqrt(jnp.sum(yT * yT, axis=0, keepdims=True))
        ne = jnp.sqrt(jnp.sum(egoT * egoT, axis=0, keepdims=True))
        return num / jnp.maximum(ny * ne, 1e-8)

    xcT, xdT = ego_cT, ego_dT
    acc_cT = jnp.zeros_like(ego_cT)
    acc_dT = jnp.zeros_like(ego_dT)
    for _ in range(N_LAYERS):
        ycT = dc * jax.lax.dot_general(
            dd * xdT, a, (((1,), (1,)), ((), ())),
            preferred_element_type=jnp.float32)
        ydT = dd * jax.lax.dot_general(
            dc * xcT, a, (((1,), (0,)), ((), ())),
            preferred_element_type=jnp.float32)
        xcT = cos_weight(ycT, ego_cT) * ycT
        xdT = cos_weight(ydT, ego_dT) * ydT
        acc_cT = acc_cT + xcT
        acc_dT = acc_dT + xdT

    circ_out[:] = jnp.transpose(acc_cT)
    dis_out[:] = jnp.transpose(acc_dT)
    tmpT = jax.lax.dot_general(
        w_ref[:], acc_cT, (((0,), (0,)), ((), ())),
        preferred_element_type=jnp.float32)
    score_out[:] = jax.lax.dot_general(
        tmpT, acc_dT, (((0,), (0,)), ((), ())),
        preferred_element_type=jnp.float32)


@functools.partial(jax.jit)
def kernel(A, circ_emb, dis_emb, re_CD):
    C, D = A.shape
    L = circ_emb.shape[1]
    out_shapes = (
        jax.ShapeDtypeStruct((C, L), jnp.float32),
        jax.ShapeDtypeStruct((D, L), jnp.float32),
        jax.ShapeDtypeStruct((C, D), jnp.float32),
    )
    return pl.pallas_call(
        _gcn_kernel,
        out_shape=out_shapes,
        compiler_params=pltpu.CompilerParams(
            vmem_limit_bytes=100 * 1024 * 1024,
        ),
    )(A, circ_emb, dis_emb, re_CD)
